# baseline (speedup 1.0000x reference)
import jax
import jax.numpy as jnp
from jax import lax
from jax.experimental import pallas as pl
from jax.experimental.pallas import tpu as pltpu
from jax.experimental.pallas import tpu_sc as plsc

B = 4096
F = 26
FP = 32
DW = 16
DD = 16
D_PAD = FP * DD
WIDE_COL = F * DD
V = 1000000
VMAIN = 999936
NROW = V // 8

_info = plsc.get_sparse_core_info()
NC, NS, L = _info.num_cores, _info.num_subcores, _info.num_lanes
NW = NC * NS
BPW = B // NW
IDX_CHUNK = 128
NCHUNK = BPW * FP // IDX_CHUNK
SLAB = 512
NSLAB = VMAIN // SLAB

_params = pltpu.CompilerParams(use_tc_tiling_on_sc=True,
                               needs_layout_passes=False)


def _detile_body(tab_t, tail128, out_hbm, slab_v, outb_v, sem_in, sem_out):
    wid = lax.axis_index("s") * NC + lax.axis_index("c")
    dd = lax.iota(jnp.int32, L)
    nbase = NSLAB // NW
    base = pl.multiple_of(wid * nbase * SLAB, 512)

    def transpose_slab(cur, ob):
        def gbody(g2, _):
            for u in range(2):
                g = g2 * 2 + u
                vloc = g * L + dd
                rows = lax.shift_right_logical(vloc, 3)
                cols = lax.shift_left(jnp.bitwise_and(vloc, 7), 4)
                for d in range(DD):
                    vec = cur[d, pl.ds(g * L, L)]
                    plsc.store_scatter(ob, [rows, cols + d], vec)
            return 0

        lax.fori_loop(0, SLAB // L // 2, gbody, 0)

    def wait_in(j):
        pltpu.make_async_copy(tab_t.at[:, pl.ds(0, SLAB)],
                              slab_v.at[j], sem_in).wait()

    def wait_out(j):
        pltpu.make_async_copy(outb_v.at[j],
                              out_hbm.at[pl.ds(0, SLAB // 8)], sem_out).wait()

    def step(s, j):
        nxt = pl.multiple_of(base + (s + 1) * SLAB, 512)
        pltpu.async_copy(tab_t.at[:, pl.ds(nxt, SLAB)],
                         slab_v.at[1 - j], sem_in)
        wait_in(j)
        transpose_slab(slab_v.at[j], outb_v.at[j])
        off = pl.multiple_of((base + s * SLAB) // 8, 64)
        pltpu.async_copy(outb_v.at[j],
                         out_hbm.at[pl.ds(off, SLAB // 8)], sem_out)

    pltpu.async_copy(tab_t.at[:, pl.ds(base, SLAB)], slab_v.at[0], sem_in)

    def pbody(p, _):
        for j in range(2):
            s = p * 2 + j

            @pl.when(p >= 1)
            def _drain():
                wait_out(j)

            step(s, j)
        return 0

    lax.fori_loop(0, (nbase - 1) // 2, pbody, 0)
    wait_out(0)
    wait_in(0)
    transpose_slab(slab_v.at[0], outb_v.at[0])
    off = pl.multiple_of((base + (nbase - 1) * SLAB) // 8, 64)
    pltpu.async_copy(outb_v.at[0],
                     out_hbm.at[pl.ds(off, SLAB // 8)], sem_out)
    wait_out(1)
    wait_out(0)

    @pl.when(wid == NW - 1)
    def _extra():
        xbase = NSLAB // NW * NW * SLAB
        pltpu.sync_copy(tab_t.at[:, pl.ds(xbase, SLAB)], slab_v.at[0])
        transpose_slab(slab_v.at[0], outb_v.at[0])
        pltpu.sync_copy(outb_v.at[0], out_hbm.at[pl.ds(xbase // 8, SLAB // 8)])
        pltpu.sync_copy(tail128, outb_v.at[0, pl.ds(0, 8)])
        pltpu.sync_copy(outb_v.at[0, pl.ds(0, 8)],
                        out_hbm.at[pl.ds(VMAIN // 8, 8)])


def _sc_detile(tab_t, tail128):
    mesh = plsc.VectorSubcoreMesh(core_axis_name="c", subcore_axis_name="s")
    fn = pl.kernel(
        _detile_body,
        mesh=mesh,
        compiler_params=_params,
        out_type=jax.ShapeDtypeStruct((NROW, 128), jnp.float32),
        scratch_types=[
            pltpu.VMEM((2, DD, SLAB), jnp.float32),
            pltpu.VMEM((2, SLAB // 8, 128), jnp.float32),
            pltpu.SemaphoreType.DMA,
            pltpu.SemaphoreType.DMA,
        ],
    )
    return fn(tab_t, tail128)


def _gather_body(xw_hbm, wsum_hbm, dlin, deep_out,
                 xw_v, idx8_v, wvals_v, chunk_v, out_v, sem_d, sem_w):
    wid = lax.axis_index("s") * NC + lax.axis_index("c")
    base = wid * BPW * FP
    dd = lax.iota(jnp.int32, L)
    pltpu.sync_copy(xw_hbm.at[pl.ds(base, BPW * FP)], xw_v)

    def i8body(j, _):
        v = xw_v[pl.ds(j * L, L)]
        idx8_v[pl.ds(j * L, L)] = lax.shift_right_logical(v, 3)
        return 0

    lax.fori_loop(0, BPW * FP // L, i8body, 0)

    dh = [pltpu.async_copy(dlin.at[idx8_v.at[pl.ds(k * IDX_CHUNK, IDX_CHUNK)]],
                           chunk_v.at[k], sem_d) for k in range(2)]

    whs = []
    for k in range(NCHUNK):
        sl = pl.ds(k * IDX_CHUNK, IDX_CHUNK)
        whs.append(pltpu.async_copy(wsum_hbm.at[xw_v.at[sl]],
                                    wvals_v.at[sl], sem_w))

    def compact(k, buf):
        def cbody(g2, _):
            for u in range(2):
                g = g2 * 2 + u
                p0 = k * IDX_CHUNK + g * L
                v16 = xw_v[pl.ds(p0, L)]
                lo = lax.shift_left(jnp.bitwise_and(v16, 7), 4)
                pos = p0 + dd
                rows = lax.shift_right_logical(pos, 5)
                cols = lax.shift_left(jnp.bitwise_and(pos, 31), 4)
                g16 = g * L + dd
                for d in range(DD):
                    vec = plsc.load_gather(buf, [g16, lo + d])
                    plsc.store_scatter(out_v, [rows, cols + d], vec)
            return 0

        lax.fori_loop(0, IDX_CHUNK // L // 2, cbody, 0)

    for k in range(NCHUNK):
        if k + 2 < NCHUNK:
            sl = pl.ds((k + 2) * IDX_CHUNK, IDX_CHUNK)
            dh.append(pltpu.async_copy(dlin.at[idx8_v.at[sl]],
                                       chunk_v.at[(k + 2) % 3], sem_d))
        dh[k].wait()
        compact(k, chunk_v.at[k % 3])

    for h in whs:
        h.wait()

    keep = dd < (FP - F + 4)

    def wbody(i, _):
        a = wvals_v[pl.ds(i * FP, L)]
        b = wvals_v[pl.ds(i * FP + L, L)]
        out_v[i, pl.ds(WIDE_COL, L)] = a + jnp.where(keep, b, 0.0)
        return 0

    lax.fori_loop(0, BPW, wbody, 0)

    pltpu.sync_copy(out_v, deep_out.at[pl.ds(wid * BPW, BPW)])


def _sc_gather(xw, wsum, dlin):
    mesh = plsc.VectorSubcoreMesh(core_axis_name="c", subcore_axis_name="s")
    fn = pl.kernel(
        _gather_body,
        mesh=mesh,
        compiler_params=_params,
        out_type=jax.ShapeDtypeStruct((B, D_PAD), jnp.float32),
        scratch_types=[
            pltpu.VMEM((BPW * FP,), jnp.int32),
            pltpu.VMEM((BPW * FP,), jnp.int32),
            pltpu.VMEM((BPW * FP,), jnp.float32),
            pltpu.VMEM((3, IDX_CHUNK, 128), jnp.float32),
            pltpu.VMEM((BPW, D_PAD), jnp.float32),
            pltpu.SemaphoreType.DMA,
            pltpu.SemaphoreType.DMA,
        ],
    )
    return fn(xw, wsum, dlin)


CHUNK_B = 1024
NB = B // CHUNK_B


def _mlp_body(deep_ref, gamma_ref, beta_ref,
              W1_ref, b1_ref, W2_ref, b2_ref, W3_ref, b3_ref, W4_ref, b4_ref,
              out_ref, sum_ref, sq_ref):
    ph = pl.program_id(0)
    c = pl.program_id(1)

    @pl.when(jnp.logical_and(ph == 0, c == 0))
    def _init():
        sum_ref[...] = jnp.zeros_like(sum_ref)
        sq_ref[...] = jnp.zeros_like(sq_ref)

    @pl.when(ph == 0)
    def _stats():
        d = deep_ref[...]
        sum_ref[...] += jnp.sum(d, axis=0, keepdims=True)
        sq_ref[...] += jnp.sum(d * d, axis=0, keepdims=True)

    @pl.when(ph == 1)
    def _mlp():
        inv_b = 1.0 / B
        mean = sum_ref[...] * inv_b
        ex2 = sq_ref[...] * inv_b
        var = ex2 - mean * mean
        scale = gamma_ref[...] * lax.rsqrt(var + 1e-5)
        shift = beta_ref[...] - mean * scale
        d = deep_ref[...]
        wide_o = jnp.sum(d[:, WIDE_COL:WIDE_COL + DW], axis=1, keepdims=True)
        h = d * scale + shift
        h = jnp.maximum(jnp.dot(h, W1_ref[...],
                                preferred_element_type=jnp.float32)
                        + b1_ref[...], 0.0)
        h = jnp.maximum(jnp.dot(h, W2_ref[...],
                                preferred_element_type=jnp.float32)
                        + b2_ref[...], 0.0)
        h = jnp.maximum(jnp.dot(h, W3_ref[...],
                                preferred_element_type=jnp.float32)
                        + b3_ref[...], 0.0)
        z = jnp.dot(h, W4_ref[...],
                    preferred_element_type=jnp.float32) + b4_ref[...]
        out_ref[...] = jax.nn.sigmoid(z + wide_o)


def _tc_mlp(deepW, gamma_p, beta_p, W1_p, b1, W2, b2, W3, b3, W4, b4):
    full = lambda shape: pl.BlockSpec(shape, lambda p, c: (0,) * len(shape))
    grid_spec = pltpu.PrefetchScalarGridSpec(
        num_scalar_prefetch=0,
        grid=(2, NB),
        in_specs=[
            pl.BlockSpec((CHUNK_B, D_PAD), lambda p, c: (c, 0)),
            full((1, D_PAD)), full((1, D_PAD)),
            full((D_PAD, 512)), full((1, 512)),
            full((512, 256)), full((1, 256)),
            full((256, 128)), full((1, 128)),
            full((128, 1)), full((1, 1)),
        ],
        out_specs=pl.BlockSpec((CHUNK_B, 1), lambda p, c: (c, 0)),
        scratch_shapes=[
            pltpu.VMEM((1, D_PAD), jnp.float32),
            pltpu.VMEM((1, D_PAD), jnp.float32),
        ],
    )
    return pl.pallas_call(
        _mlp_body,
        grid_spec=grid_spec,
        out_shape=jax.ShapeDtypeStruct((B, 1), jnp.float32),
    )(deepW, gamma_p, beta_p,
      W1_p, b1.reshape(1, -1), W2, b2.reshape(1, -1),
      W3, b3.reshape(1, -1), W4, b4.reshape(1, -1))


def kernel(x, wide_table, deep_table, gamma, beta,
           W1, b1, W2, b2, W3, b3, W4, b4):
    pad_cols = D_PAD - F * DD
    x32 = x.astype(jnp.int32)
    xw = jnp.concatenate([x32, x32[:, :FP - F]], axis=1).reshape(B * FP)
    wsum = jnp.sum(wide_table, axis=1)
    tail128 = deep_table[VMAIN:].reshape(8, 128)
    dlin = _sc_detile(deep_table.T, tail128)
    deepW = _sc_gather(xw, wsum, dlin)
    gamma_p = jnp.pad(gamma, (0, pad_cols)).reshape(1, D_PAD)
    beta_p = jnp.pad(beta, (0, pad_cols)).reshape(1, D_PAD)
    W1_p = jnp.pad(W1, ((0, pad_cols), (0, 0)))
    return _tc_mlp(deepW, gamma_p, beta_p, W1_p,
                   b1, W2, b2, W3, b3, W4, b4)


# speedup vs baseline: 1.0404x; 1.0404x over previous
import jax
import jax.numpy as jnp
from jax import lax
from jax.experimental import pallas as pl
from jax.experimental.pallas import tpu as pltpu
from jax.experimental.pallas import tpu_sc as plsc

B = 4096
F = 26
FP = 32
DW = 16
DD = 16
D_PAD = FP * DD
WIDE_COL = F * DD
V = 1000000
VMAIN = 999936
NROW = V // 8

_info = plsc.get_sparse_core_info()
NC, NS, L = _info.num_cores, _info.num_subcores, _info.num_lanes
NW = NC * NS
BPW = B // NW
IDX_CHUNK = 128
NCHUNK = BPW * FP // IDX_CHUNK
SLAB = 512
NSLAB = VMAIN // SLAB

_params = pltpu.CompilerParams(use_tc_tiling_on_sc=True,
                               needs_layout_passes=False)


def _detile_body(tab_t, tail128, out_hbm, slab_v, outb_v, sem_in, sem_out):
    wid = lax.axis_index("s") * NC + lax.axis_index("c")
    dd = lax.iota(jnp.int32, L)
    nbase = NSLAB // NW
    base = pl.multiple_of(wid * nbase * SLAB, 512)

    def transpose_slab(cur, ob):
        def gbody(g2, _):
            for u in range(2):
                g = g2 * 2 + u
                vloc = g * L + dd
                rows = lax.shift_right_logical(vloc, 3)
                cols = lax.shift_left(jnp.bitwise_and(vloc, 7), 4)
                for d in range(DD):
                    vec = cur[d, pl.ds(g * L, L)]
                    plsc.store_scatter(ob, [rows, cols + d], vec)
            return 0

        lax.fori_loop(0, SLAB // L // 2, gbody, 0)

    def wait_in(j):
        pltpu.make_async_copy(tab_t.at[:, pl.ds(0, SLAB)],
                              slab_v.at[j], sem_in).wait()

    def wait_out(j):
        pltpu.make_async_copy(outb_v.at[j],
                              out_hbm.at[pl.ds(0, SLAB // 8)], sem_out).wait()

    def step(s, j):
        nxt = pl.multiple_of(base + (s + 1) * SLAB, 512)
        pltpu.async_copy(tab_t.at[:, pl.ds(nxt, SLAB)],
                         slab_v.at[1 - j], sem_in)
        wait_in(j)
        transpose_slab(slab_v.at[j], outb_v.at[j])
        off = pl.multiple_of((base + s * SLAB) // 8, 64)
        pltpu.async_copy(outb_v.at[j],
                         out_hbm.at[pl.ds(off, SLAB // 8)], sem_out)

    pltpu.async_copy(tab_t.at[:, pl.ds(base, SLAB)], slab_v.at[0], sem_in)

    def pbody(p, _):
        for j in range(2):
            s = p * 2 + j

            @pl.when(p >= 1)
            def _drain():
                wait_out(j)

            step(s, j)
        return 0

    lax.fori_loop(0, (nbase - 1) // 2, pbody, 0)
    wait_out(0)
    wait_in(0)
    transpose_slab(slab_v.at[0], outb_v.at[0])
    off = pl.multiple_of((base + (nbase - 1) * SLAB) // 8, 64)
    pltpu.async_copy(outb_v.at[0],
                     out_hbm.at[pl.ds(off, SLAB // 8)], sem_out)
    wait_out(1)
    wait_out(0)

    @pl.when(wid == NW - 1)
    def _extra():
        xbase = NSLAB // NW * NW * SLAB
        pltpu.sync_copy(tab_t.at[:, pl.ds(xbase, SLAB)], slab_v.at[0])
        transpose_slab(slab_v.at[0], outb_v.at[0])
        pltpu.sync_copy(outb_v.at[0], out_hbm.at[pl.ds(xbase // 8, SLAB // 8)])
        pltpu.sync_copy(tail128, outb_v.at[0, pl.ds(0, 8)])
        pltpu.sync_copy(outb_v.at[0, pl.ds(0, 8)],
                        out_hbm.at[pl.ds(VMAIN // 8, 8)])


def _sc_detile(tab_t, tail128):
    mesh = plsc.VectorSubcoreMesh(core_axis_name="c", subcore_axis_name="s")
    fn = pl.kernel(
        _detile_body,
        mesh=mesh,
        compiler_params=_params,
        out_type=jax.ShapeDtypeStruct((NROW, 128), jnp.float32),
        scratch_types=[
            pltpu.VMEM((2, DD, SLAB), jnp.float32),
            pltpu.VMEM((2, SLAB // 8, 128), jnp.float32),
            pltpu.SemaphoreType.DMA,
            pltpu.SemaphoreType.DMA,
        ],
    )
    return fn(tab_t, tail128)


NCHUNK_D = BPW * F // IDX_CHUNK
MAGIC26 = 20165


def _gather_body(xd_hbm, xw_hbm, wsum_hbm, dlin, deep_out,
                 xd_v, xw_v, idx8_v, wvals_v, chunk_v, out_v, sem_d, sem_w):
    wid = lax.axis_index("s") * NC + lax.axis_index("c")
    dd = lax.iota(jnp.int32, L)
    pltpu.sync_copy(xd_hbm.at[pl.ds(wid * BPW * F, BPW * F)], xd_v)
    pltpu.sync_copy(xw_hbm.at[pl.ds(wid * BPW * FP, BPW * FP)], xw_v)

    def i8body(j, _):
        v = xd_v[pl.ds(j * L, L)]
        idx8_v[pl.ds(j * L, L)] = lax.shift_right_logical(v, 3)
        return 0

    lax.fori_loop(0, BPW * F // L, i8body, 0)

    dh = [pltpu.async_copy(dlin.at[idx8_v.at[pl.ds(k * IDX_CHUNK, IDX_CHUNK)]],
                           chunk_v.at[k], sem_d) for k in range(2)]

    whs = []
    for k in range(NCHUNK):
        sl = pl.ds(k * IDX_CHUNK, IDX_CHUNK)
        whs.append(pltpu.async_copy(wsum_hbm.at[xw_v.at[sl]],
                                    wvals_v.at[sl], sem_w))

    def compact(k, buf):
        def cbody(g2, _):
            for u in range(2):
                g = g2 * 2 + u
                p0 = k * IDX_CHUNK + g * L
                v16 = xd_v[pl.ds(p0, L)]
                lo = lax.shift_left(jnp.bitwise_and(v16, 7), 4)
                pos = p0 + dd
                rows = lax.shift_right_logical(pos * MAGIC26, 19)
                cols = lax.shift_left(pos - rows * F, 4)
                g16 = g * L + dd
                for d in range(DD):
                    vec = plsc.load_gather(buf, [g16, lo + d])
                    plsc.store_scatter(out_v, [rows, cols + d], vec)
            return 0

        lax.fori_loop(0, IDX_CHUNK // L // 2, cbody, 0)

    for k in range(NCHUNK_D):
        if k + 2 < NCHUNK_D:
            sl = pl.ds((k + 2) * IDX_CHUNK, IDX_CHUNK)
            dh.append(pltpu.async_copy(dlin.at[idx8_v.at[sl]],
                                       chunk_v.at[(k + 2) % 3], sem_d))
        dh[k].wait()
        compact(k, chunk_v.at[k % 3])

    for h in whs:
        h.wait()

    keep = dd < (FP - F + 4)
    zeros = jnp.zeros((L,), jnp.float32)

    def wbody(i, _):
        a = wvals_v[pl.ds(i * FP, L)]
        b = wvals_v[pl.ds(i * FP + L, L)]
        out_v[i, pl.ds(WIDE_COL, L)] = a + jnp.where(keep, b, 0.0)
        for sl in range(F + 1, FP):
            out_v[i, pl.ds(sl * DD, L)] = zeros
        return 0

    lax.fori_loop(0, BPW, wbody, 0)

    pltpu.sync_copy(out_v, deep_out.at[pl.ds(wid * BPW, BPW)])


def _sc_gather(xd, xw, wsum, dlin):
    mesh = plsc.VectorSubcoreMesh(core_axis_name="c", subcore_axis_name="s")
    fn = pl.kernel(
        _gather_body,
        mesh=mesh,
        compiler_params=_params,
        out_type=jax.ShapeDtypeStruct((B, D_PAD), jnp.float32),
        scratch_types=[
            pltpu.VMEM((BPW * F,), jnp.int32),
            pltpu.VMEM((BPW * FP,), jnp.int32),
            pltpu.VMEM((BPW * F,), jnp.int32),
            pltpu.VMEM((BPW * FP,), jnp.float32),
            pltpu.VMEM((3, IDX_CHUNK, 128), jnp.float32),
            pltpu.VMEM((BPW, D_PAD), jnp.float32),
            pltpu.SemaphoreType.DMA,
            pltpu.SemaphoreType.DMA,
        ],
    )
    return fn(xd, xw, wsum, dlin)


CHUNK_B = 1024
NB = B // CHUNK_B


def _mlp_body(deep_ref, gamma_ref, beta_ref,
              W1_ref, b1_ref, W2_ref, b2_ref, W3_ref, b3_ref, W4_ref, b4_ref,
              out_ref, sum_ref, sq_ref):
    ph = pl.program_id(0)
    c = pl.program_id(1)

    @pl.when(jnp.logical_and(ph == 0, c == 0))
    def _init():
        sum_ref[...] = jnp.zeros_like(sum_ref)
        sq_ref[...] = jnp.zeros_like(sq_ref)

    @pl.when(ph == 0)
    def _stats():
        d = deep_ref[...]
        sum_ref[...] += jnp.sum(d, axis=0, keepdims=True)
        sq_ref[...] += jnp.sum(d * d, axis=0, keepdims=True)

    @pl.when(ph == 1)
    def _mlp():
        inv_b = 1.0 / B
        mean = sum_ref[...] * inv_b
        ex2 = sq_ref[...] * inv_b
        var = ex2 - mean * mean
        scale = gamma_ref[...] * lax.rsqrt(var + 1e-5)
        shift = beta_ref[...] - mean * scale
        d = deep_ref[...]
        wide_o = jnp.sum(d[:, WIDE_COL:WIDE_COL + DW], axis=1, keepdims=True)
        h = d * scale + shift
        h = jnp.maximum(jnp.dot(h, W1_ref[...],
                                preferred_element_type=jnp.float32)
                        + b1_ref[...], 0.0)
        h = jnp.maximum(jnp.dot(h, W2_ref[...],
                                preferred_element_type=jnp.float32)
                        + b2_ref[...], 0.0)
        h = jnp.maximum(jnp.dot(h, W3_ref[...],
                                preferred_element_type=jnp.float32)
                        + b3_ref[...], 0.0)
        z = jnp.dot(h, W4_ref[...],
                    preferred_element_type=jnp.float32) + b4_ref[...]
        out_ref[...] = jax.nn.sigmoid(z + wide_o)


def _tc_mlp(deepW, gamma_p, beta_p, W1_p, b1, W2, b2, W3, b3, W4, b4):
    full = lambda shape: pl.BlockSpec(shape, lambda p, c: (0,) * len(shape))
    grid_spec = pltpu.PrefetchScalarGridSpec(
        num_scalar_prefetch=0,
        grid=(2, NB),
        in_specs=[
            pl.BlockSpec((CHUNK_B, D_PAD), lambda p, c: (c, 0)),
            full((1, D_PAD)), full((1, D_PAD)),
            full((D_PAD, 512)), full((1, 512)),
            full((512, 256)), full((1, 256)),
            full((256, 128)), full((1, 128)),
            full((128, 1)), full((1, 1)),
        ],
        out_specs=pl.BlockSpec((CHUNK_B, 1), lambda p, c: (c, 0)),
        scratch_shapes=[
            pltpu.VMEM((1, D_PAD), jnp.float32),
            pltpu.VMEM((1, D_PAD), jnp.float32),
        ],
    )
    return pl.pallas_call(
        _mlp_body,
        grid_spec=grid_spec,
        out_shape=jax.ShapeDtypeStruct((B, 1), jnp.float32),
    )(deepW, gamma_p, beta_p,
      W1_p, b1.reshape(1, -1), W2, b2.reshape(1, -1),
      W3, b3.reshape(1, -1), W4, b4.reshape(1, -1))


def kernel(x, wide_table, deep_table, gamma, beta,
           W1, b1, W2, b2, W3, b3, W4, b4):
    pad_cols = D_PAD - F * DD
    x32 = x.astype(jnp.int32)
    xd = x32.reshape(B * F)
    xw = jnp.concatenate([x32, x32[:, :FP - F]], axis=1).reshape(B * FP)
    wsum = jnp.sum(wide_table, axis=1)
    tail128 = deep_table[VMAIN:].reshape(8, 128)
    dlin = _sc_detile(deep_table.T, tail128)
    deepW = _sc_gather(xd, xw, wsum, dlin)
    gamma_p = jnp.pad(gamma, (0, pad_cols)).reshape(1, D_PAD)
    beta_p = jnp.pad(beta, (0, pad_cols)).reshape(1, D_PAD)
    W1_p = jnp.pad(W1, ((0, pad_cols), (0, 0)))
    return _tc_mlp(deepW, gamma_p, beta_p, W1_p,
                   b1, W2, b2, W3, b3, W4, b4)
